# baseline (speedup 1.0000x reference)
import functools

import jax
import jax.numpy as jnp
from jax import lax
from jax.experimental import pallas as pl
from jax.experimental.pallas import tpu as pltpu
from jax.experimental.pallas import tpu_sc as plsc

N_NODES = 10000
D = 128
E = 320000
NC = 2
NS = 16
L = 16
NW = NC * NS

EPW = 9984
EXTRA = E - NW * EPW
S = 128
CPW = EPW // S

ROWS_PER_TILE = 624
ROWS_REM = N_NODES - NS * ROWS_PER_TILE


def _scale_rows(rows_ref, w_ref, g0, g1):
    bidx = [jnp.full((L,), r, jnp.int32) for r in range(L)]
    dnums = lax.GatherDimensionNumbers(
        offset_dims=(), collapsed_slice_dims=(0,), start_index_map=(0,))

    def group_body(g, _):
        wg = w_ref[pl.ds(g * L, L)]
        r0 = g * L
        for r in range(L):
            wv = lax.gather(wg, bidx[r][:, None], dnums, slice_sizes=(1,),
                            mode=lax.GatherScatterMode.PROMISE_IN_BOUNDS)
            row = r0 + r
            for j in range(D // L):
                rows_ref[row, pl.ds(j * L, L)] = (
                    rows_ref[row, pl.ds(j * L, L)] * wv)
        return 0

    lax.fori_loop(g0, g1, group_body, 0)


def _sc_body(x_hbm, epk_hbm, w_hbm, zeros_hbm, out_hbm,
             pkA, wbA, dstSA, rowsA,
             pkB, wbB, dstSB, rowsB,
             pkC, wbC, dstSC, rowsC,
             semIA, semGA, semSA, semIB, semGB, semSB, semIC, semGC, semSC,
             acc_sh):
    c = lax.axis_index("c")
    s = lax.axis_index("s")
    wid = s * NC + c

    A = dict(pk=pkA, wb=wbA, dstS=dstSA, rows=rowsA,
             semI=semIA, semG=semGA, semS=semSA)
    B = dict(pk=pkB, wb=wbB, dstS=dstSB, rows=rowsB,
             semI=semIB, semG=semGB, semS=semSB)
    C = dict(pk=pkC, wb=wbC, dstS=dstSC, rows=rowsC,
             semI=semIC, semG=semGC, semS=semSC)

    pltpu.sync_copy(zeros_hbm.at[pl.ds(0, ROWS_PER_TILE)],
                    acc_sh.at[pl.ds(s * ROWS_PER_TILE, ROWS_PER_TILE)])

    @pl.when(s == NS - 1)
    def _zero_rem():
        pltpu.sync_copy(zeros_hbm.at[pl.ds(0, ROWS_REM)],
                        acc_sh.at[pl.ds(NS * ROWS_PER_TILE, ROWS_REM)])

    plsc.subcore_barrier()

    base_e = wid * EPW

    def issue_idx(off, bufs):
        pltpu.async_copy(epk_hbm.at[:, pl.ds(off, S)], bufs["pk"], bufs["semI"])
        pltpu.async_copy(w_hbm.at[pl.ds(off, S)], bufs["wb"], bufs["semI"])

    def wait_idx(off, bufs):
        pltpu.make_async_copy(epk_hbm.at[:, pl.ds(off, S)], bufs["pk"],
                              bufs["semI"]).wait()
        pltpu.make_async_copy(w_hbm.at[pl.ds(off, S)], bufs["wb"],
                              bufs["semI"]).wait()

    def issue_gather(bufs):
        pltpu.async_copy(x_hbm.at[bufs["pk"].at[1]], bufs["rows"], bufs["semG"])

    def wait_gather(bufs):
        pltpu.make_async_copy(x_hbm.at[bufs["pk"].at[1]], bufs["rows"],
                              bufs["semG"]).wait()

    H = S // 2

    def issue_scatter_half(bufs, j):
        pltpu.async_copy(bufs["rows"].at[pl.ds(j * H, H)],
                         acc_sh.at[bufs["dstS"].at[j]],
                         bufs["semS"], add=True)

    def wait_scatter(bufs):
        for j in range(2):
            pltpu.make_async_copy(bufs["rows"].at[pl.ds(j * H, H)],
                                  acc_sh.at[bufs["dstS"].at[j]],
                                  bufs["semS"]).wait()

    def free_idx_slot(bufs):
        for j in range(2):
            for k in range(4):
                bufs["dstS"][j, pl.ds(k * L, L)] = (
                    bufs["pk"][0, pl.ds(j * H + k * L, L)])

    def process(n_off, p, q, first, do_next, do_idx3):
        if do_next:
            wait_idx(n_off + S, q)
            if not first:
                wait_scatter(q)
            issue_gather(q)
        wait_gather(p)
        free_idx_slot(p)
        _scale_rows(p["rows"], p["wb"], 0, S // (2 * L))
        issue_scatter_half(p, 0)
        _scale_rows(p["rows"], p["wb"], S // (2 * L), S // L)
        issue_scatter_half(p, 1)
        if do_idx3:
            issue_idx(n_off + 3 * S, p)

    issue_idx(base_e, A)
    issue_idx(base_e + S, B)
    issue_idx(base_e + 2 * S, C)
    wait_idx(base_e, A)
    issue_gather(A)

    process(base_e, A, B, first=True, do_next=True, do_idx3=True)
    process(base_e + S, B, C, first=True, do_next=True, do_idx3=True)

    def triple_body(i, _):
        n_off = base_e + (2 + 3 * i) * S
        process(n_off, C, A, first=False, do_next=True, do_idx3=True)
        process(n_off + S, A, B, first=False, do_next=True, do_idx3=True)
        process(n_off + 2 * S, B, C, first=False, do_next=True, do_idx3=True)
        return 0

    lax.fori_loop(0, (CPW - 6) // 3, triple_body, 0)

    process(base_e + (CPW - 4) * S, C, A,
            first=False, do_next=True, do_idx3=True)
    process(base_e + (CPW - 3) * S, A, B,
            first=False, do_next=True, do_idx3=False)
    process(base_e + (CPW - 2) * S, B, C,
            first=False, do_next=True, do_idx3=False)
    process(base_e + (CPW - 1) * S, C, A,
            first=False, do_next=False, do_idx3=False)
    wait_scatter(A)
    wait_scatter(B)
    wait_scatter(C)

    @pl.when(wid < EXTRA // S)
    def _extra():
        off = NW * EPW + wid * S
        pltpu.sync_copy(epk_hbm.at[:, pl.ds(off, S)], pkA)
        pltpu.sync_copy(w_hbm.at[pl.ds(off, S)], wbA)
        free_idx_slot(A)
        pltpu.async_copy(x_hbm.at[pkA.at[1]], rowsA, semGA).wait()
        _scale_rows(rowsA, wbA, 0, S // L)
        issue_scatter_half(A, 0)
        issue_scatter_half(A, 1)
        wait_scatter(A)

    plsc.subcore_barrier()

    pltpu.sync_copy(
        acc_sh.at[pl.ds(s * ROWS_PER_TILE, ROWS_PER_TILE)],
        out_hbm.at[c, pl.ds(s * ROWS_PER_TILE, ROWS_PER_TILE)])

    @pl.when(s == NS - 1)
    def _dump_rem():
        pltpu.sync_copy(
            acc_sh.at[pl.ds(NS * ROWS_PER_TILE, ROWS_REM)],
            out_hbm.at[c, pl.ds(NS * ROWS_PER_TILE, ROWS_REM)])


def _sc_partials(x, epk, w, zeros):
    mesh = plsc.VectorSubcoreMesh(
        core_axis_name="c", subcore_axis_name="s", num_cores=NC, num_subcores=NS)
    pbuf = lambda: pltpu.VMEM((2, S), jnp.int32)
    sbuf = lambda: pltpu.VMEM((2, S // 2), jnp.int32)
    fbuf = lambda: pltpu.VMEM((S,), jnp.float32)
    rbuf = lambda: pltpu.VMEM((S, D), jnp.float32)
    return pl.kernel(
        _sc_body,
        out_type=jax.ShapeDtypeStruct((NC, N_NODES, D), jnp.float32),
        mesh=mesh,
        scratch_types=[
            pbuf(), fbuf(), sbuf(), rbuf(),
            pbuf(), fbuf(), sbuf(), rbuf(),
            pbuf(), fbuf(), sbuf(), rbuf(),
            pltpu.SemaphoreType.DMA, pltpu.SemaphoreType.DMA,
            pltpu.SemaphoreType.DMA, pltpu.SemaphoreType.DMA,
            pltpu.SemaphoreType.DMA, pltpu.SemaphoreType.DMA,
            pltpu.SemaphoreType.DMA, pltpu.SemaphoreType.DMA,
            pltpu.SemaphoreType.DMA,
            pltpu.VMEM_SHARED((N_NODES, D), jnp.float32),
        ],
    )(x, epk, w, zeros)


BN = 2000


def _tc_body(scal_ref, p_ref, o_ref):
    a = scal_ref[0, 0] * (p_ref[0] + p_ref[1])
    o_ref[...] = jnp.where(a > 0, a, jnp.exp(jnp.minimum(a, 0.0)) - 1.0)


def _tc_combine(scalar, partials):
    return pl.pallas_call(
        _tc_body,
        grid=(N_NODES // BN,),
        in_specs=[
            pl.BlockSpec((1, 1), lambda i: (0, 0), memory_space=pltpu.SMEM),
            pl.BlockSpec((NC, BN, D), lambda i: (0, i, 0)),
        ],
        out_specs=pl.BlockSpec((BN, D), lambda i: (i, 0)),
        out_shape=jax.ShapeDtypeStruct((N_NODES, D), jnp.float32),
    )(scalar, partials)


def kernel(x, edge_index, adj_values, scalar):
    epk = edge_index.astype(jnp.int32)
    zeros = jnp.zeros((ROWS_PER_TILE, D), jnp.float32)
    partials = _sc_partials(x, epk, adj_values, zeros)
    return _tc_combine(scalar.reshape(1, 1).astype(jnp.float32), partials)


# speedup vs baseline: 1.0190x; 1.0190x over previous
import functools

import jax
import jax.numpy as jnp
from jax import lax
from jax.experimental import pallas as pl
from jax.experimental.pallas import tpu as pltpu
from jax.experimental.pallas import tpu_sc as plsc

N_NODES = 10000
D = 128
E = 320000
NC = 2
NS = 16
L = 16
NW = NC * NS

EPW = 9984
EXTRA = E - NW * EPW
S = 128
CPW = EPW // S

ROWS_PER_TILE = 624
ROWS_REM = N_NODES - NS * ROWS_PER_TILE


def _scale_rows(rows_ref, w_ref, nrows):
    bidx = [jnp.full((L,), r, jnp.int32) for r in range(L)]
    dnums = lax.GatherDimensionNumbers(
        offset_dims=(), collapsed_slice_dims=(0,), start_index_map=(0,))

    def group_body(g, _):
        wg = w_ref[pl.ds(g * L, L)]
        r0 = g * L
        for r in range(L):
            wv = lax.gather(wg, bidx[r][:, None], dnums, slice_sizes=(1,),
                            mode=lax.GatherScatterMode.PROMISE_IN_BOUNDS)
            row = r0 + r
            for j in range(D // L):
                rows_ref[row, pl.ds(j * L, L)] = (
                    rows_ref[row, pl.ds(j * L, L)] * wv)
        return 0

    lax.fori_loop(0, nrows // L, group_body, 0)


def _sc_body(x_hbm, epk_hbm, w_hbm, scal_hbm, zeros_hbm, out_hbm,
             pkA, wbA, dstSA, rowsA,
             pkB, wbB, dstSB, rowsB,
             pkC, wbC, dstSC, rowsC,
             wS, scbuf,
             semIA, semGA, semSA, semIB, semGB, semSB, semIC, semGC, semSC,
             acc_sh):
    c = lax.axis_index("c")
    s = lax.axis_index("s")
    wid = s * NC + c

    A = dict(pk=pkA, wb=wbA, dstS=dstSA, rows=rowsA,
             semI=semIA, semG=semGA, semS=semSA)
    B = dict(pk=pkB, wb=wbB, dstS=dstSB, rows=rowsB,
             semI=semIB, semG=semGB, semS=semSB)
    C = dict(pk=pkC, wb=wbC, dstS=dstSC, rows=rowsC,
             semI=semIC, semG=semGC, semS=semSC)

    base_e = wid * EPW

    stag = lax.fori_loop(0, s * 30, lambda i, a: a + 1, jnp.int32(0))
    dstSA[0, pl.ds(0, L)] = jnp.zeros((L,), jnp.int32) + stag

    def issue_idx(off, bufs):
        pltpu.async_copy(epk_hbm.at[:, pl.ds(off, S)], bufs["pk"], bufs["semI"])
        pltpu.async_copy(w_hbm.at[pl.ds(off, S)], bufs["wb"], bufs["semI"])

    def wait_idx(off, bufs):
        pltpu.make_async_copy(epk_hbm.at[:, pl.ds(off, S)], bufs["pk"],
                              bufs["semI"]).wait()
        pltpu.make_async_copy(w_hbm.at[pl.ds(off, S)], bufs["wb"],
                              bufs["semI"]).wait()

    def issue_gather(bufs):
        pltpu.async_copy(x_hbm.at[bufs["pk"].at[1]], bufs["rows"], bufs["semG"])

    def wait_gather(bufs):
        pltpu.make_async_copy(x_hbm.at[bufs["pk"].at[1]], bufs["rows"],
                              bufs["semG"]).wait()

    def issue_scatter(bufs):
        pltpu.async_copy(bufs["rows"], acc_sh.at[bufs["dstS"].at[0]],
                         bufs["semS"], add=True)

    def wait_scatter(bufs):
        pltpu.make_async_copy(bufs["rows"], acc_sh.at[bufs["dstS"].at[0]],
                              bufs["semS"]).wait()

    def free_idx_slot(bufs):
        for k in range(8):
            bufs["dstS"][0, pl.ds(k * L, L)] = bufs["pk"][0, pl.ds(k * L, L)]
        for k in range(8):
            wS[pl.ds(k * L, L)] = sv * bufs["wb"][pl.ds(k * L, L)]

    def process(n_off, p, q, first, do_next, do_idx3):
        if do_next:
            wait_idx(n_off + S, q)
            if not first:
                wait_scatter(q)
            issue_gather(q)
        wait_gather(p)
        free_idx_slot(p)
        if do_idx3:
            issue_idx(n_off + 3 * S, p)
        _scale_rows(p["rows"], wS, S)
        issue_scatter(p)

    issue_idx(base_e, A)
    issue_idx(base_e + S, B)
    issue_idx(base_e + 2 * S, C)
    pltpu.sync_copy(scal_hbm, scbuf)
    wait_idx(base_e, A)
    issue_gather(A)

    pltpu.sync_copy(zeros_hbm.at[pl.ds(0, ROWS_PER_TILE)],
                    acc_sh.at[pl.ds(s * ROWS_PER_TILE, ROWS_PER_TILE)])

    @pl.when(s == NS - 1)
    def _zero_rem():
        pltpu.sync_copy(zeros_hbm.at[pl.ds(0, ROWS_REM)],
                        acc_sh.at[pl.ds(NS * ROWS_PER_TILE, ROWS_REM)])

    plsc.subcore_barrier()
    sv = scbuf[pl.ds(0, L)]

    process(base_e, A, B, first=True, do_next=True, do_idx3=True)
    process(base_e + S, B, C, first=True, do_next=True, do_idx3=True)

    def triple_body(i, _):
        n_off = base_e + (2 + 3 * i) * S
        process(n_off, C, A, first=False, do_next=True, do_idx3=True)
        process(n_off + S, A, B, first=False, do_next=True, do_idx3=True)
        process(n_off + 2 * S, B, C, first=False, do_next=True, do_idx3=True)
        return 0

    lax.fori_loop(0, (CPW - 6) // 3, triple_body, 0)

    process(base_e + (CPW - 4) * S, C, A,
            first=False, do_next=True, do_idx3=True)
    process(base_e + (CPW - 3) * S, A, B,
            first=False, do_next=True, do_idx3=False)
    process(base_e + (CPW - 2) * S, B, C,
            first=False, do_next=True, do_idx3=False)
    process(base_e + (CPW - 1) * S, C, A,
            first=False, do_next=False, do_idx3=False)
    wait_scatter(A)
    wait_scatter(B)
    wait_scatter(C)

    @pl.when(wid < EXTRA // S)
    def _extra():
        off = NW * EPW + wid * S
        pltpu.sync_copy(epk_hbm.at[:, pl.ds(off, S)], pkA)
        pltpu.sync_copy(w_hbm.at[pl.ds(off, S)], wbA)
        free_idx_slot(A)
        pltpu.async_copy(x_hbm.at[pkA.at[1]], rowsA, semGA).wait()
        _scale_rows(rowsA, wS, S)
        pltpu.async_copy(rowsA, acc_sh.at[dstSA.at[0]], semSA, add=True).wait()

    plsc.subcore_barrier()

    pltpu.sync_copy(
        acc_sh.at[pl.ds(s * ROWS_PER_TILE, ROWS_PER_TILE)],
        out_hbm.at[c, pl.ds(s * ROWS_PER_TILE, ROWS_PER_TILE)])

    @pl.when(s == NS - 1)
    def _dump_rem():
        pltpu.sync_copy(
            acc_sh.at[pl.ds(NS * ROWS_PER_TILE, ROWS_REM)],
            out_hbm.at[c, pl.ds(NS * ROWS_PER_TILE, ROWS_REM)])


def _sc_partials(x, epk, w, scal, zeros):
    mesh = plsc.VectorSubcoreMesh(
        core_axis_name="c", subcore_axis_name="s", num_cores=NC, num_subcores=NS)
    pbuf = lambda: pltpu.VMEM((2, S), jnp.int32)
    sbuf = lambda: pltpu.VMEM((1, S), jnp.int32)
    fbuf = lambda: pltpu.VMEM((S,), jnp.float32)
    rbuf = lambda: pltpu.VMEM((S, D), jnp.float32)
    return pl.kernel(
        _sc_body,
        out_type=jax.ShapeDtypeStruct((NC, N_NODES, D), jnp.float32),
        mesh=mesh,
        scratch_types=[
            pbuf(), fbuf(), sbuf(), rbuf(),
            pbuf(), fbuf(), sbuf(), rbuf(),
            pbuf(), fbuf(), sbuf(), rbuf(),
            fbuf(),
            pltpu.VMEM((L,), jnp.float32),
            pltpu.SemaphoreType.DMA, pltpu.SemaphoreType.DMA,
            pltpu.SemaphoreType.DMA, pltpu.SemaphoreType.DMA,
            pltpu.SemaphoreType.DMA, pltpu.SemaphoreType.DMA,
            pltpu.SemaphoreType.DMA, pltpu.SemaphoreType.DMA,
            pltpu.SemaphoreType.DMA,
            pltpu.VMEM_SHARED((N_NODES, D), jnp.float32),
        ],
    )(x, epk, w, scal, zeros)


BN = 2000


def _tc_body(p_ref, o_ref):
    a = p_ref[0] + p_ref[1]
    o_ref[...] = jnp.where(a > 0, a, jnp.exp(jnp.minimum(a, 0.0)) - 1.0)


def _tc_combine(partials):
    return pl.pallas_call(
        _tc_body,
        grid=(N_NODES // BN,),
        in_specs=[pl.BlockSpec((NC, BN, D), lambda i: (0, i, 0))],
        out_specs=pl.BlockSpec((BN, D), lambda i: (i, 0)),
        out_shape=jax.ShapeDtypeStruct((N_NODES, D), jnp.float32),
    )(partials)


def kernel(x, edge_index, adj_values, scalar):
    epk = edge_index.astype(jnp.int32)
    scal16 = jnp.broadcast_to(scalar.astype(jnp.float32), (L,))
    zeros = jnp.zeros((ROWS_PER_TILE, D), jnp.float32)
    partials = _sc_partials(x, epk, adj_values, scal16, zeros)
    return _tc_combine(partials)
